# baseline (speedup 1.0000x reference)
import functools

import jax
import jax.numpy as jnp
from jax.experimental import pallas as pl
from jax.experimental.pallas import tpu as pltpu

N_TOKENS = 4096
N_CODES = 16384
DIM = 2048
BM = 2048
BN = 256


def _vq_argmin_body(x_ref, e_ref, x2_ref, e2_ref, idx_ref,
                    accv_ref, acci_ref, winv_ref, wini_ref):
    j = pl.program_id(1)
    dot = jax.lax.dot_general(
        x_ref[...], e_ref[...], (((1,), (1,)), ((), ())),
        preferred_element_type=jnp.float32)
    d = (x2_ref[...] + e2_ref[...]) - 2.0 * dot
    bmin = jnp.min(d, axis=1, keepdims=True)
    cols = jax.lax.broadcasted_iota(jnp.int32, d.shape, 1)
    bidx = jnp.min(jnp.where(d == bmin, cols, N_CODES),
                   axis=1, keepdims=True) + j * BN

    win_start = j % 5 == 0
    win_end = jnp.logical_or(jnp.logical_and(j % 5 == 4, j < 60), j == 63)

    @pl.when(j == 0)
    def _():
        accv_ref[...] = jnp.full_like(accv_ref[...], jnp.inf)
        acci_ref[...] = jnp.zeros_like(acci_ref[...])

    @pl.when(win_start)
    def _():
        winv_ref[...] = bmin
        wini_ref[...] = bidx

    @pl.when(jnp.logical_not(win_start))
    def _():
        upd = bmin < winv_ref[...]
        winv_ref[...] = jnp.where(upd, bmin, winv_ref[...])
        wini_ref[...] = jnp.where(upd, bidx, wini_ref[...])

    @pl.when(win_end)
    def _():
        upd = winv_ref[...] < accv_ref[...]
        accv_ref[...] = jnp.where(
            upd,
            winv_ref[...].astype(jnp.bfloat16).astype(jnp.float32),
            accv_ref[...])
        acci_ref[...] = jnp.where(upd, wini_ref[...], acci_ref[...])

    @pl.when(j == pl.num_programs(1) - 1)
    def _():
        idx_ref[...] = acci_ref[...]


@jax.jit
def kernel(x, embedding):
    x2 = jnp.sum(x ** 2, axis=1, keepdims=True)
    e2 = jnp.sum(embedding ** 2, axis=1)[None, :]

    grid = (N_TOKENS // BM, N_CODES // BN)
    idx2d = pl.pallas_call(
        _vq_argmin_body,
        grid=grid,
        in_specs=[
            pl.BlockSpec((BM, DIM), lambda i, j: (i, 0)),
            pl.BlockSpec((BN, DIM), lambda i, j: (j, 0)),
            pl.BlockSpec((BM, 1), lambda i, j: (i, 0)),
            pl.BlockSpec((1, BN), lambda i, j: (0, j)),
        ],
        out_specs=pl.BlockSpec((BM, 1), lambda i, j: (i, 0)),
        out_shape=jax.ShapeDtypeStruct((N_TOKENS, 1), jnp.int32),
        scratch_shapes=[
            pltpu.VMEM((BM, 1), jnp.float32),
            pltpu.VMEM((BM, 1), jnp.int32),
            pltpu.VMEM((BM, 1), jnp.float32),
            pltpu.VMEM((BM, 1), jnp.int32),
        ],
        compiler_params=pltpu.CompilerParams(
            dimension_semantics=("arbitrary", "arbitrary")),
    )(x, embedding, x2, e2)

    indices = idx2d[:, 0]
    z_q = jnp.take(embedding, indices, axis=0)
    return (z_q, indices)


# speedup vs baseline: 1.1043x; 1.1043x over previous
import functools

import jax
import jax.numpy as jnp
from jax.experimental import pallas as pl
from jax.experimental.pallas import tpu as pltpu

N_TOKENS = 4096
N_CODES = 16384
DIM = 2048
BM = 2048
BN = 256


def _vq_argmin_body(x_ref, e_ref, x2_ref, e2_ref, idx_ref,
                    accv_ref, acci_ref, winm_ref, winb_ref):
    j = pl.program_id(1)
    dot = jax.lax.dot_general(
        x_ref[...], e_ref[...], (((1,), (1,)), ((), ())),
        preferred_element_type=jnp.float32)
    d = (x2_ref[...] + e2_ref[...]) - 2.0 * dot

    win_start = j % 5 == 0
    win_end = jnp.logical_or(jnp.logical_and(j % 5 == 4, j < 60), j == 63)

    @pl.when(j == 0)
    def _():
        accv_ref[...] = jnp.full_like(accv_ref[...], jnp.inf)
        acci_ref[...] = jnp.zeros_like(acci_ref[...])

    @pl.when(win_start)
    def _():
        winm_ref[...] = d
        winb_ref[...] = jnp.full_like(winb_ref[...], j)

    @pl.when(jnp.logical_not(win_start))
    def _():
        upd = d < winm_ref[...]
        winm_ref[...] = jnp.where(upd, d, winm_ref[...])
        winb_ref[...] = jnp.where(upd, j, winb_ref[...])

    @pl.when(win_end)
    def _():
        winm = winm_ref[...]
        wmin = jnp.min(winm, axis=1, keepdims=True)
        cols = jax.lax.broadcasted_iota(jnp.int32, winm.shape, 1)
        g = winb_ref[...] * BN + cols
        widx = jnp.min(jnp.where(winm == wmin, g, N_CODES),
                       axis=1, keepdims=True)
        upd = wmin < accv_ref[...]
        accv_ref[...] = jnp.where(
            upd, wmin.astype(jnp.bfloat16).astype(jnp.float32),
            accv_ref[...])
        acci_ref[...] = jnp.where(upd, widx, acci_ref[...])

    @pl.when(j == pl.num_programs(1) - 1)
    def _():
        idx_ref[...] = acci_ref[...]


@jax.jit
def kernel(x, embedding):
    x2 = jnp.sum(x ** 2, axis=1, keepdims=True)
    e2 = jnp.sum(embedding ** 2, axis=1)[None, :]

    grid = (N_TOKENS // BM, N_CODES // BN)
    idx2d = pl.pallas_call(
        _vq_argmin_body,
        grid=grid,
        in_specs=[
            pl.BlockSpec((BM, DIM), lambda i, j: (i, 0)),
            pl.BlockSpec((BN, DIM), lambda i, j: (j, 0)),
            pl.BlockSpec((BM, 1), lambda i, j: (i, 0)),
            pl.BlockSpec((1, BN), lambda i, j: (0, j)),
        ],
        out_specs=pl.BlockSpec((BM, 1), lambda i, j: (i, 0)),
        out_shape=jax.ShapeDtypeStruct((N_TOKENS, 1), jnp.int32),
        scratch_shapes=[
            pltpu.VMEM((BM, 1), jnp.float32),
            pltpu.VMEM((BM, 1), jnp.int32),
            pltpu.VMEM((BM, BN), jnp.float32),
            pltpu.VMEM((BM, BN), jnp.int32),
        ],
        compiler_params=pltpu.CompilerParams(
            dimension_semantics=("parallel", "arbitrary")),
    )(x, embedding, x2, e2)

    indices = idx2d[:, 0]
    z_q = jnp.take(embedding, indices, axis=0)
    return (z_q, indices)


# speedup vs baseline: 1.1689x; 1.0585x over previous
import functools

import jax
import jax.numpy as jnp
from jax.experimental import pallas as pl
from jax.experimental.pallas import tpu as pltpu

N_TOKENS = 4096
N_CODES = 16384
DIM = 2048
BM = 2048
BN = 256


def _vq_argmin_body(x_ref, e_ref, x2_ref, idx_ref,
                    accv_ref, acci_ref, winm_ref, winb_ref):
    j = pl.program_id(1)
    e_blk = e_ref[...]
    dot = jax.lax.dot_general(
        x_ref[...], e_blk, (((1,), (1,)), ((), ())),
        preferred_element_type=jnp.float32)
    e2 = jnp.sum(e_blk * e_blk, axis=1)[None, :]
    d = (x2_ref[...] + e2) - 2.0 * dot

    win_start = j % 5 == 0
    win_end = jnp.logical_or(jnp.logical_and(j % 5 == 4, j < 60), j == 63)

    @pl.when(j == 0)
    def _():
        accv_ref[...] = jnp.full_like(accv_ref[...], jnp.inf)
        acci_ref[...] = jnp.zeros_like(acci_ref[...])

    @pl.when(win_start)
    def _():
        winm_ref[...] = d
        winb_ref[...] = jnp.full_like(winb_ref[...], j)

    @pl.when(jnp.logical_not(win_start))
    def _():
        upd = d < winm_ref[...]
        winm_ref[...] = jnp.where(upd, d, winm_ref[...])
        winb_ref[...] = jnp.where(upd, j, winb_ref[...])

    @pl.when(win_end)
    def _():
        winm = winm_ref[...]
        wmin = jnp.min(winm, axis=1, keepdims=True)
        cols = jax.lax.broadcasted_iota(jnp.int32, winm.shape, 1)
        g = winb_ref[...] * BN + cols
        widx = jnp.min(jnp.where(winm == wmin, g, N_CODES),
                       axis=1, keepdims=True)
        upd = wmin < accv_ref[...]
        accv_ref[...] = jnp.where(
            upd, wmin.astype(jnp.bfloat16).astype(jnp.float32),
            accv_ref[...])
        acci_ref[...] = jnp.where(upd, widx, acci_ref[...])

    @pl.when(j == pl.num_programs(1) - 1)
    def _():
        idx_ref[...] = acci_ref[...]


@jax.jit
def kernel(x, embedding):
    x2 = jnp.sum(x ** 2, axis=1, keepdims=True)

    grid = (N_TOKENS // BM, N_CODES // BN)
    idx2d = pl.pallas_call(
        _vq_argmin_body,
        grid=grid,
        in_specs=[
            pl.BlockSpec((BM, DIM), lambda i, j: (i, 0)),
            pl.BlockSpec((BN, DIM), lambda i, j: (j, 0)),
            pl.BlockSpec((BM, 1), lambda i, j: (i, 0)),
        ],
        out_specs=pl.BlockSpec((BM, 1), lambda i, j: (i, 0)),
        out_shape=jax.ShapeDtypeStruct((N_TOKENS, 1), jnp.int32),
        scratch_shapes=[
            pltpu.VMEM((BM, 1), jnp.float32),
            pltpu.VMEM((BM, 1), jnp.int32),
            pltpu.VMEM((BM, BN), jnp.float32),
            pltpu.VMEM((BM, BN), jnp.int32),
        ],
        compiler_params=pltpu.CompilerParams(
            dimension_semantics=("parallel", "arbitrary")),
    )(x, embedding, x2)

    indices = idx2d[:, 0]
    z_q = jnp.take(embedding, indices, axis=0)
    return (z_q, indices)
